# baseline (speedup 1.0000x reference)
import functools

import jax
import jax.numpy as jnp
from jax import lax
from jax.experimental import pallas as pl
from jax.experimental.pallas import tpu as pltpu
from jax.experimental.pallas import tpu_sc as plsc

EMBED = 64
NCHUNK = 4


def kernel(next_id, next_category, next_subcategory, id_table, category_table,
           subcategory_table):
    B = next_id.shape[0]
    next_id = next_id.astype(jnp.int32)
    next_category = next_category.astype(jnp.int32)
    next_subcategory = next_subcategory.astype(jnp.int32)

    info = plsc.get_sparse_core_info()
    nw = info.num_cores * info.num_subcores
    b_per_w = B // nw
    chunk = b_per_w // NCHUNK

    mesh = plsc.VectorSubcoreMesh(core_axis_name="c", subcore_axis_name="s")

    @functools.partial(
        pl.kernel,
        mesh=mesh,
        out_type=jax.ShapeDtypeStruct((B, 3 * EMBED), jnp.float32),
        compiler_params=pltpu.CompilerParams(use_tc_tiling_on_sc=False),
        scratch_types=[
            pltpu.VMEM((b_per_w,), jnp.int32),
            pltpu.VMEM((b_per_w,), jnp.int32),
            pltpu.VMEM((b_per_w,), jnp.int32),
            [[pltpu.VMEM((chunk, EMBED), jnp.float32) for _ in range(2)]
             for _ in range(3)],
            [[pltpu.SemaphoreType.DMA for _ in range(2)] for _ in range(3)],
            [[pltpu.SemaphoreType.DMA for _ in range(2)] for _ in range(3)],
        ],
    )
    def gather_concat(id_idx_hbm, cat_idx_hbm, sub_idx_hbm, id_tab, cat_tab,
                      sub_tab, out_hbm, idx0, idx1, idx2, rows, gsem, wsem):
        wid = lax.axis_index("s") * info.num_cores + lax.axis_index("c")
        base = wid * b_per_w
        pltpu.sync_copy(id_idx_hbm.at[pl.ds(base, b_per_w)], idx0)
        pltpu.sync_copy(cat_idx_hbm.at[pl.ds(base, b_per_w)], idx1)
        pltpu.sync_copy(sub_idx_hbm.at[pl.ds(base, b_per_w)], idx2)
        tabs = (id_tab, cat_tab, sub_tab)
        idxs = (idx0, idx1, idx2)

        def fire_gathers(c):
            p = c % 2
            return [
                pltpu.async_copy(
                    tabs[t].at[idxs[t].at[pl.ds(c * chunk, chunk)]],
                    rows[t][p], gsem[t][p])
                for t in range(3)
            ]

        def fire_writes(c):
            p = c % 2
            return [
                pltpu.async_copy(
                    rows[t][p],
                    out_hbm.at[pl.ds(base + c * chunk, chunk),
                               pl.ds(t * EMBED, EMBED)],
                    wsem[t][p])
                for t in range(3)
            ]

        gathers = fire_gathers(0)
        writes = [None, None]
        for c in range(NCHUNK):
            p = c % 2
            if c + 1 < NCHUNK:
                if writes[1 - p] is not None:
                    for w in writes[1 - p]:
                        w.wait()
                    writes[1 - p] = None
                next_gathers = fire_gathers(c + 1)
            for g in gathers:
                g.wait()
            writes[p] = fire_writes(c)
            if c + 1 < NCHUNK:
                gathers = next_gathers
        for ws in writes:
            if ws is not None:
                for w in ws:
                    w.wait()

    return gather_concat(next_id, next_category, next_subcategory, id_table,
                         category_table, subcategory_table)
